# baseline (speedup 1.0000x reference)
import functools

import jax
import jax.numpy as jnp
from jax import lax
from jax.experimental import pallas as pl
from jax.experimental.pallas import tpu as pltpu
from jax.experimental.pallas import tpu_sc as plsc

VOCAB = 100000
D = 64
B = 4096
HIST = 50
HIST_PAD = 56
NC, NS = 2, 16
NW = NC * NS
ITEMS_PER_W = B // NW
NBUF = 16
PREFETCH = 13
LANES = 16
DJ = D // LANES
ITEM_F = HIST * D

C1 = 0.24999940826684283
C3 = -0.02082532326072556
C5 = 0.0020537565075574096
C7 = -0.00016932519223054887


@functools.partial(
    pl.kernel,
    mesh=plsc.VectorSubcoreMesh(core_axis_name="c", subcore_axis_name="s"),
    out_type=jax.ShapeDtypeStruct((B * HIST * D,), jnp.float32),
    scratch_types=[
        pltpu.VMEM((ITEMS_PER_W * HIST_PAD,), jnp.int32),
        pltpu.VMEM((NBUF, HIST_PAD, D), jnp.float32),
        pltpu.VMEM((4, ITEM_F), jnp.float32),
    ] + [pltpu.SemaphoreType.DMA] * 20,
    compiler_params=pltpu.CompilerParams(use_tc_tiling_on_sc=False),
)
def _sc_kernel(feat_hbm, table_hbm, out_hbm, idx_v, rows_v, obuf_v, *sems):
    gsems = sems[:NBUF]
    osems = sems[NBUF:]
    cid = lax.axis_index("c")
    sid = lax.axis_index("s")
    wid = sid * NC + cid
    item0 = wid * ITEMS_PER_W

    def gather_start(t, buf):
        pltpu.async_copy(
            table_hbm.at[idx_v.at[pl.ds(t * HIST_PAD, HIST_PAD)]],
            rows_v.at[buf], gsems[buf])

    def gather_wait(buf):
        pltpu.make_async_copy(
            table_hbm.at[idx_v.at[pl.ds(0, HIST_PAD)]],
            rows_v.at[buf], gsems[buf]).wait()

    def wb_start(t, ob):
        pltpu.async_copy(
            obuf_v.at[ob],
            out_hbm.at[pl.ds((item0 + t) * ITEM_F, ITEM_F)],
            osems[ob])

    def wb_wait(ob):
        pltpu.make_async_copy(
            obuf_v.at[ob],
            out_hbm.at[pl.ds(item0 * ITEM_F, ITEM_F)],
            osems[ob]).wait()

    pltpu.sync_copy(
        feat_hbm.at[pl.ds(item0 * HIST_PAD, ITEMS_PER_W * HIST_PAD)], idx_v)
    for p in range(PREFETCH):
        gather_start(p, p)

    def ring_body(i, carry):
        for b in range(NBUF):
            t = NBUF * i + b
            ob = b % 4

            @pl.when(t >= 4)
            def _():
                wb_wait(ob)

            @pl.when(t + PREFETCH < ITEMS_PER_W)
            def _():
                gather_start(t + PREFETCH, (b + PREFETCH) % NBUF)

            gather_wait(b)

            zero = jnp.zeros((LANES,), jnp.float32)

            def p1(l, acc):
                return tuple(
                    acc[j] + jnp.abs(rows_v[b, l, pl.ds(j * LANES, LANES)])
                    for j in range(DJ))

            acc = lax.fori_loop(0, HIST, p1, (zero,) * DJ, unroll=2)
            rn = tuple(1.0 / jnp.maximum(acc[j], 1e-12) for j in range(DJ))

            def p2(l, cc, rn=rn):
                for j in range(DJ):
                    e = rows_v[b, l, pl.ds(j * LANES, LANES)]
                    x = e * rn[j]
                    x2 = x * x
                    p = C7 * x2 + C5
                    p = p * x2 + C3
                    p = p * x2 + C1
                    y = x * p + 0.5
                    obuf_v[ob, pl.ds(l * D + j * LANES, LANES)] = y
                return cc

            lax.fori_loop(0, HIST, p2, 0, unroll=2)
            wb_start(t, ob)
        return carry

    lax.fori_loop(0, ITEMS_PER_W // NBUF, ring_body, 0)
    for ob in range(4):
        wb_wait(ob)


def kernel(features, emb_table):
    feats = features.astype(jnp.int32)
    feats_p = jnp.concatenate([feats, feats[:, :HIST_PAD - HIST]], axis=1)
    out = _sc_kernel(feats_p.reshape(-1), emb_table)
    return out.reshape(B, HIST, D)
